# Initial kernel scaffold; baseline (speedup 1.0000x reference)
import jax
import jax.numpy as jnp
from jax.experimental import pallas as pl


def kernel(x_batch, LOS_batch, template_edge_index, emb_table, W1a, b1a, ln1_g, ln1_b, W1b, b1b, W2a, b2a, ln2_g, ln2_b, W2b, b2b, Wih, Whh, bih, bhh, Wc1, bc1, Wc2, bc2):
    raise NotImplementedError("write your pallas kernel here")


# speedup vs baseline: 1.7369x; 1.7369x over previous
import functools

import jax
import jax.numpy as jnp
from jax.experimental import pallas as pl
from jax.experimental.pallas import tpu as pltpu

B = 1024
V = 72
N = 36
EMB = 128
H = 256
GRU_H = 256
COL_DIM = 100
NODES = 2 * B * N
NGRAPH = 2 * B
MAX_LOS = 37.0
RB = 288
GB = 8
GRID = NODES // RB


def _matmul_kernel(h_ref, w_ref, o_ref):
    o_ref[...] = jnp.dot(h_ref[...], w_ref[...],
                         preferred_element_type=jnp.float32)


def _matmul(h, w):
    din = h.shape[1]
    return pl.pallas_call(
        _matmul_kernel,
        grid=(GRID,),
        in_specs=[
            pl.BlockSpec((RB, din), lambda i: (i, 0)),
            pl.BlockSpec((din, H), lambda i: (0, 0)),
        ],
        out_specs=pl.BlockSpec((RB, H), lambda i: (i, 0)),
        out_shape=jax.ShapeDtypeStruct((NODES, H), jnp.float32),
    )(h, w)


def _mlpb_kernel(y_ref, agg_ref, ba_ref, g_ref, lb_ref, wb_ref, bb_ref,
                 a_ref, h_ref, p_ref):
    z = y_ref[...] + agg_ref[...] + ba_ref[...]
    mu = jnp.mean(z, axis=-1, keepdims=True)
    zc = z - mu
    var = jnp.mean(zc * zc, axis=-1, keepdims=True)
    zn = zc * jax.lax.rsqrt(var + 1e-5) * g_ref[...] + lb_ref[...]
    hrelu = jnp.maximum(zn, 0.0)
    out = jnp.dot(hrelu, wb_ref[...],
                  preferred_element_type=jnp.float32) + bb_ref[...]
    h_ref[...] = out
    p_ref[...] = jnp.dot(a_ref[...], out, preferred_element_type=jnp.float32)


def _mlpb(y, agg, ba, g, lb, wb, bb, pool_mat):
    return pl.pallas_call(
        _mlpb_kernel,
        grid=(GRID,),
        in_specs=[
            pl.BlockSpec((RB, H), lambda i: (i, 0)),
            pl.BlockSpec((RB, H), lambda i: (i, 0)),
            pl.BlockSpec((1, H), lambda i: (0, 0)),
            pl.BlockSpec((1, H), lambda i: (0, 0)),
            pl.BlockSpec((1, H), lambda i: (0, 0)),
            pl.BlockSpec((H, H), lambda i: (0, 0)),
            pl.BlockSpec((1, H), lambda i: (0, 0)),
            pl.BlockSpec((GB, RB), lambda i: (0, 0)),
        ],
        out_specs=[
            pl.BlockSpec((RB, H), lambda i: (i, 0)),
            pl.BlockSpec((GB, H), lambda i: (i, 0)),
        ],
        out_shape=[
            jax.ShapeDtypeStruct((NODES, H), jnp.float32),
            jax.ShapeDtypeStruct((NGRAPH, H), jnp.float32),
        ],
    )(y, agg, ba, g, lb, wb, bb, pool_mat)


def _gru_kernel(xad_ref, xdis_ref, wih_ref, whh_ref, bih_ref, bhh_ref,
                wc1_ref, bc1_ref, wc2_ref, o_ref):
    bih = bih_ref[...]
    bhh = bhh_ref[...]
    gi = jnp.dot(xad_ref[...], wih_ref[...],
                 preferred_element_type=jnp.float32) + bih
    ir, iz, inn = gi[:, :H], gi[:, H:2 * H], gi[:, 2 * H:]
    hr, hz, hn = bhh[:, :H], bhh[:, H:2 * H], bhh[:, 2 * H:]
    r = jax.nn.sigmoid(ir + hr)
    z = jax.nn.sigmoid(iz + hz)
    n = jnp.tanh(inn + r * hn)
    hh = (1.0 - z) * n
    gi2 = jnp.dot(xdis_ref[...], wih_ref[...],
                  preferred_element_type=jnp.float32) + bih
    gh2 = jnp.dot(hh, whh_ref[...],
                  preferred_element_type=jnp.float32) + bhh
    r2 = jax.nn.sigmoid(gi2[:, :H] + gh2[:, :H])
    z2 = jax.nn.sigmoid(gi2[:, H:2 * H] + gh2[:, H:2 * H])
    n2 = jnp.tanh(gi2[:, 2 * H:] + r2 * gh2[:, 2 * H:])
    hh2 = (1.0 - z2) * n2 + z2 * hh
    c = jnp.maximum(jnp.dot(hh2, wc1_ref[...],
                            preferred_element_type=jnp.float32)
                    + bc1_ref[...], 0.0)
    o_ref[...] = jnp.dot(c, wc2_ref[...], preferred_element_type=jnp.float32)


def _gru_head(gin, wih_t, whh_t, bih, bhh, wc1, bc1, wc2p):
    rb = 256
    grid = B // rb
    out = pl.pallas_call(
        _gru_kernel,
        grid=(grid,),
        in_specs=[
            pl.BlockSpec((rb, 3 * H), lambda i: (i, 0)),
            pl.BlockSpec((rb, 3 * H), lambda i: (i + grid, 0)),
            pl.BlockSpec((3 * H, 3 * H), lambda i: (0, 0)),
            pl.BlockSpec((H, 3 * H), lambda i: (0, 0)),
            pl.BlockSpec((1, 3 * H), lambda i: (0, 0)),
            pl.BlockSpec((1, 3 * H), lambda i: (0, 0)),
            pl.BlockSpec((H, 2 * H), lambda i: (0, 0)),
            pl.BlockSpec((1, 2 * H), lambda i: (0, 0)),
            pl.BlockSpec((2 * H, 128), lambda i: (0, 0)),
        ],
        out_specs=pl.BlockSpec((rb, 128), lambda i: (i, 0)),
        out_shape=jax.ShapeDtypeStruct((B, 128), jnp.float32),
    )(gin, gin, wih_t, whh_t, bih, bhh, wc1, bc1, wc2p)
    return out[:, :1]


def kernel(x_batch, LOS_batch, template_edge_index, emb_table, W1a, b1a,
           ln1_g, ln1_b, W1b, b1b, W2a, b2a, ln2_g, ln2_b, W2b, b2b,
           Wih, Whh, bih, bhh, Wc1, bc1, Wc2, bc2):
    g = jnp.arange(NGRAPH, dtype=jnp.int32)
    is_dis = (g >= B).astype(jnp.int32)
    brow = g - is_dis * B
    cols = is_dis[:, None] * 36 + jnp.arange(N, dtype=jnp.int32)[None, :]
    flat = x_batch[brow[:, None], cols] + cols * COL_DIM
    nidx = flat.reshape(NODES)
    los_node = jnp.broadcast_to(
        (LOS_batch[brow] / MAX_LOS)[:, None], (NGRAPH, N)).reshape(NODES)

    src = template_edge_index[0]
    dst = template_edge_index[1]

    emb_h = jnp.take(emb_table, nidx, axis=0)
    h0 = jnp.concatenate(
        [emb_h, los_node[:, None],
         jnp.zeros((NODES, 7), jnp.float32)], axis=1)
    w1a_p = jnp.concatenate(
        [W1a, jnp.zeros((7, H), jnp.float32)], axis=0)

    pool_mat = jnp.repeat(jnp.eye(GB, dtype=jnp.float32), N, axis=1)

    def edge_agg(y):
        return jnp.zeros_like(y).at[dst].add(jnp.take(y, src, axis=0))

    biases2 = [(b1a, ln1_g, ln1_b, W1b, b1b),
               (b2a, ln2_g, ln2_b, W2b, b2b),
               (b2a, ln2_g, ln2_b, W2b, b2b)]
    h = h0
    wa = w1a_p
    pooled = []
    for li in range(3):
        y = _matmul(h, wa)
        agg = edge_agg(y)
        ba, lg, lb, wb, bb = biases2[li]
        h, p = _mlpb(y, agg, ba.reshape(1, H), lg.reshape(1, H),
                     lb.reshape(1, H), wb, bb.reshape(1, H), pool_mat)
        pooled.append(p)
        wa = W2a
    gin = jnp.concatenate(pooled, axis=1)

    wc2p = jnp.concatenate(
        [Wc2, jnp.zeros((2 * H, 127), jnp.float32)], axis=1)
    out = _gru_head(gin, Wih.T, Whh.T, bih.reshape(1, 3 * H),
                    bhh.reshape(1, 3 * H), Wc1, bc1.reshape(1, 2 * H), wc2p)
    return out + bc2[0]


# speedup vs baseline: 1.8291x; 1.0531x over previous
import functools

import jax
import jax.numpy as jnp
from jax import lax
from jax.experimental import pallas as pl
from jax.experimental.pallas import tpu as pltpu
from jax.experimental.pallas import tpu_sc as plsc

B = 1024
V = 72
N = 36
EMB = 128
H = 256
GRU_H = 256
COL_DIM = 100
NODES = 2 * B * N
NGRAPH = 2 * B
MAX_LOS = 37.0
RB = 288
GB = 8
GRID = NODES // RB


E = 8 * NODES
EPT = E // 16
EB = 128
NB = EPT // EB
SW = 16
NSLC = H // SW
RPT = NODES // 16

_MESH = plsc.VectorSubcoreMesh(
    core_axis_name="c", subcore_axis_name="s", num_cores=2, num_subcores=16)


@functools.partial(
    pl.kernel,
    out_type=jax.ShapeDtypeStruct((NODES, EMB), jnp.float32),
    mesh=_MESH,
    scratch_types=[
        pltpu.VMEM((18, 128), jnp.int32),
        pltpu.VMEM((128, EMB), jnp.float32),
        pltpu.VMEM((128, EMB), jnp.float32),
        pltpu.SemaphoreType.DMA,
        pltpu.SemaphoreType.DMA,
    ],
)
def _emb_gather(table, idx3, out, idxv, buf0, buf1, sem0, sem1):
    c = lax.axis_index("c")
    s = lax.axis_index("s")
    w = s * 2 + c
    pltpu.sync_copy(idx3.at[w], idxv)
    base = w * 2304
    bufs = (buf0, buf1)
    sems = (sem0, sem1)
    descs = [None] * 18
    descs[0] = pltpu.async_copy(table.at[idxv.at[0]], buf0, sem0)
    for j in range(18):
        if j + 1 < 18:
            descs[j + 1] = pltpu.async_copy(
                table.at[idxv.at[j + 1]], bufs[(j + 1) % 2], sems[(j + 1) % 2])
        descs[j].wait()
        pltpu.sync_copy(bufs[j % 2], out.at[pl.ds(base + j * 128, 128), :])


NCH = 36
CROWS = 2048
BCAP = 145
TRASH = CROWS
EPW = E // 32


@functools.partial(
    pl.kernel,
    out_type=[
        jax.ShapeDtypeStruct((NCH, 32, BCAP * 128), jnp.int32),
        jax.ShapeDtypeStruct((32, 48), jnp.int32),
    ],
    mesh=_MESH,
    compiler_params=pltpu.CompilerParams(use_tc_tiling_on_sc=False,
                                        needs_layout_passes=False),
    scratch_types=[
        pltpu.VMEM((EPW,), jnp.int32),
        pltpu.VMEM((EPW,), jnp.int32),
        pltpu.VMEM((NCH * 272,), jnp.int32),
        pltpu.VMEM((48,), jnp.int32),
    ],
)
def _bucketize(src2, dst2, pbkt, nblk, srcv, dstv, stage, nbv):
    c = lax.axis_index("c")
    s = lax.axis_index("s")
    w = s * 2 + c
    pltpu.sync_copy(src2.at[w], srcv)
    pltpu.sync_copy(dst2.at[w], dstv)

    zero = jnp.zeros((), jnp.int32)
    init = (tuple(zero for _ in range(NCH)), tuple(zero for _ in range(NCH)))

    def _scan(i, carry):
        curs, fbs = carry
        sv = srcv[pl.ds(i * 16, 16)]
        dv = dstv[pl.ds(i * 16, 16)]
        bkt = lax.shift_right_logical(dv, 11)
        packed = jnp.bitwise_or(
            lax.shift_left(sv, 12), jnp.bitwise_and(dv, 2047))
        new_curs = []
        new_fbs = []
        for ch in range(NCH):
            m = bkt == ch
            cnt = jnp.sum(m.astype(jnp.int32))
            cur = curs[ch]
            fb = fbs[ch]
            plsc.store_compressed(stage.at[pl.ds(ch * 272 + cur, 16)],
                                  packed, mask=m)
            cur = cur + cnt
            do_flush = cur >= 128

            @pl.when(do_flush)
            def _():
                pltpu.sync_copy(stage.at[pl.ds(ch * 272, 128)],
                                pbkt.at[ch, w, pl.ds(fb * 128, 128)])
                tail = stage[pl.ds(ch * 272 + 128, 16)]
                stage[pl.ds(ch * 272, 16)] = tail
            cur = jnp.where(do_flush, cur - 128, cur)
            fb = jnp.where(do_flush, fb + 1, fb)
            new_curs.append(cur)
            new_fbs.append(fb)
        return tuple(new_curs), tuple(new_fbs)

    curs, fbs = lax.fori_loop(0, EPW // 16, _scan, init)

    trash = jnp.full((16,), TRASH, jnp.int32)
    lane = lax.broadcasted_iota(jnp.int32, (16,), 0)
    vecs = [jnp.zeros((16,), jnp.int32) for _ in range(3)]
    for ch in range(NCH):
        cur = curs[ch]
        fb = fbs[ch]
        for k in range(8):
            stage[pl.ds(ch * 272 + cur + k * 16, 16)] = trash

        @pl.when(cur > 0)
        def _():
            pltpu.sync_copy(stage.at[pl.ds(ch * 272, 128)],
                            pbkt.at[ch, w, pl.ds(fb * 128, 128)])
        nbt = jnp.broadcast_to(fb + jnp.where(cur > 0, 1, 0), (16,))
        g = ch // 16
        vecs[g] = jnp.where(lane == ch - g * 16, nbt, vecs[g])
    for g in range(3):
        nbv[pl.ds(g * 16, 16)] = vecs[g]
    pltpu.sync_copy(nbv, nblk.at[w])


@functools.partial(
    pl.kernel,
    out_type=jax.ShapeDtypeStruct((NODES, H), jnp.float32),
    mesh=_MESH,
    compiler_params=pltpu.CompilerParams(use_tc_tiling_on_sc=False,
                                        needs_layout_passes=False),
    scratch_types=[
        pltpu.VMEM((128,), jnp.int32),
        pltpu.VMEM((128,), jnp.int32),
        pltpu.VMEM((128,), jnp.int32),
        pltpu.VMEM((48,), jnp.int32),
        pltpu.VMEM((48,), jnp.int32),
        pltpu.VMEM((128, H), jnp.float32),
        pltpu.VMEM((128, H), jnp.float32),
        pltpu.VMEM_SHARED((CROWS + 8, H), jnp.float32),
        pltpu.SemaphoreType.DMA,
    ],
)
def _edge_agg(y, pbkt, nblk, agg, pbuf, sidx, didx, nbv0, nbv1, zb, buf,
              acc, sem):
    c = lax.axis_index("c")
    s = lax.axis_index("s")

    def _zb(q, carry):
        zb[q >> 4, pl.ds((q & 15) * 16, 16)] = jnp.zeros((16,), jnp.float32)
        return carry
    lax.fori_loop(0, 2048, _zb, 0)
    pltpu.sync_copy(nblk.at[2 * s], nbv0)
    pltpu.sync_copy(nblk.at[2 * s + 1], nbv1)

    for k in range(NCH // 2):
        ck = 2 * k + c
        pltpu.sync_copy(zb, acc.at[pl.ds(s * 128, 128), :])

        @pl.when(s == 0)
        def _():
            pltpu.sync_copy(zb.at[pl.ds(0, 8), :], acc.at[pl.ds(CROWS, 8), :])
        plsc.subcore_barrier()

        lane = lax.broadcasted_iota(jnp.int32, (16,), 0)
        for w_off in range(2):
            nbr = nbv0 if w_off == 0 else nbv1
            va = nbr[pl.ds(0, 16)]
            vb = nbr[pl.ds(16, 16)]
            vc = nbr[pl.ds(32, 16)]
            vsel = jnp.where(jnp.broadcast_to(ck < 16, (16,)), va,
                             jnp.where(jnp.broadcast_to(ck < 32, (16,)),
                                       vb, vc))
            nb = jnp.max(jnp.where(lane == ck % 16, vsel, 0))

            def _block(b, carry):
                pltpu.sync_copy(
                    pbkt.at[ck, 2 * s + w_off, pl.ds(b * 128, 128)], pbuf)
                for q in range(8):
                    v = pbuf[pl.ds(q * 16, 16)]
                    sidx[pl.ds(q * 16, 16)] = lax.shift_right_logical(v, 12)
                    didx[pl.ds(q * 16, 16)] = jnp.bitwise_and(v, 4095)
                pltpu.async_copy(y.at[sidx], buf, sem).wait()
                pltpu.sync_copy(buf, acc.at[didx], add=True)
                return carry
            lax.fori_loop(0, nb, _block, 0)
        plsc.subcore_barrier()
        pltpu.sync_copy(acc.at[pl.ds(s * 128, 128), :],
                        agg.at[pl.ds(ck * CROWS + s * 128, 128), :])


def _matmul_kernel(h_ref, w_ref, o_ref):
    o_ref[...] = jnp.dot(h_ref[...], w_ref[...],
                         preferred_element_type=jnp.float32)


def _matmul(h, w):
    din = h.shape[1]
    return pl.pallas_call(
        _matmul_kernel,
        grid=(GRID,),
        in_specs=[
            pl.BlockSpec((RB, din), lambda i: (i, 0)),
            pl.BlockSpec((din, H), lambda i: (0, 0)),
        ],
        out_specs=pl.BlockSpec((RB, H), lambda i: (i, 0)),
        out_shape=jax.ShapeDtypeStruct((NODES, H), jnp.float32),
    )(h, w)


def _matmul1_kernel(h_ref, los_ref, w_ref, wl_ref, o_ref):
    o_ref[...] = (jnp.dot(h_ref[...], w_ref[...],
                          preferred_element_type=jnp.float32)
                  + los_ref[...] * wl_ref[...])


def _matmul1(h, los, w, wl):
    return pl.pallas_call(
        _matmul1_kernel,
        grid=(GRID,),
        in_specs=[
            pl.BlockSpec((RB, EMB), lambda i: (i, 0)),
            pl.BlockSpec((RB, 1), lambda i: (i, 0)),
            pl.BlockSpec((EMB, H), lambda i: (0, 0)),
            pl.BlockSpec((1, H), lambda i: (0, 0)),
        ],
        out_specs=pl.BlockSpec((RB, H), lambda i: (i, 0)),
        out_shape=jax.ShapeDtypeStruct((NODES, H), jnp.float32),
    )(h, los, w, wl)


def _mlpb_kernel(y_ref, agg_ref, ba_ref, g_ref, lb_ref, wb_ref, bb_ref,
                 a_ref, h_ref, p_ref):
    z = y_ref[...] + agg_ref[...] + ba_ref[...]
    mu = jnp.mean(z, axis=-1, keepdims=True)
    zc = z - mu
    var = jnp.mean(zc * zc, axis=-1, keepdims=True)
    zn = zc * jax.lax.rsqrt(var + 1e-5) * g_ref[...] + lb_ref[...]
    hrelu = jnp.maximum(zn, 0.0)
    out = jnp.dot(hrelu, wb_ref[...],
                  preferred_element_type=jnp.float32) + bb_ref[...]
    h_ref[...] = out
    p_ref[...] = jnp.dot(a_ref[...], out, preferred_element_type=jnp.float32)


def _mlpb(y, agg, ba, g, lb, wb, bb, pool_mat):
    return pl.pallas_call(
        _mlpb_kernel,
        grid=(GRID,),
        in_specs=[
            pl.BlockSpec((RB, H), lambda i: (i, 0)),
            pl.BlockSpec((RB, H), lambda i: (i, 0)),
            pl.BlockSpec((1, H), lambda i: (0, 0)),
            pl.BlockSpec((1, H), lambda i: (0, 0)),
            pl.BlockSpec((1, H), lambda i: (0, 0)),
            pl.BlockSpec((H, H), lambda i: (0, 0)),
            pl.BlockSpec((1, H), lambda i: (0, 0)),
            pl.BlockSpec((GB, RB), lambda i: (0, 0)),
        ],
        out_specs=[
            pl.BlockSpec((RB, H), lambda i: (i, 0)),
            pl.BlockSpec((GB, H), lambda i: (i, 0)),
        ],
        out_shape=[
            jax.ShapeDtypeStruct((NODES, H), jnp.float32),
            jax.ShapeDtypeStruct((NGRAPH, H), jnp.float32),
        ],
    )(y, agg, ba, g, lb, wb, bb, pool_mat)


def _gru_kernel(xad_ref, xdis_ref, wih_ref, whh_ref, bih_ref, bhh_ref,
                wc1_ref, bc1_ref, wc2_ref, o_ref):
    bih = bih_ref[...]
    bhh = bhh_ref[...]
    gi = jnp.dot(xad_ref[...], wih_ref[...],
                 preferred_element_type=jnp.float32) + bih
    ir, iz, inn = gi[:, :H], gi[:, H:2 * H], gi[:, 2 * H:]
    hr, hz, hn = bhh[:, :H], bhh[:, H:2 * H], bhh[:, 2 * H:]
    r = jax.nn.sigmoid(ir + hr)
    z = jax.nn.sigmoid(iz + hz)
    n = jnp.tanh(inn + r * hn)
    hh = (1.0 - z) * n
    gi2 = jnp.dot(xdis_ref[...], wih_ref[...],
                  preferred_element_type=jnp.float32) + bih
    gh2 = jnp.dot(hh, whh_ref[...],
                  preferred_element_type=jnp.float32) + bhh
    r2 = jax.nn.sigmoid(gi2[:, :H] + gh2[:, :H])
    z2 = jax.nn.sigmoid(gi2[:, H:2 * H] + gh2[:, H:2 * H])
    n2 = jnp.tanh(gi2[:, 2 * H:] + r2 * gh2[:, 2 * H:])
    hh2 = (1.0 - z2) * n2 + z2 * hh
    c = jnp.maximum(jnp.dot(hh2, wc1_ref[...],
                            preferred_element_type=jnp.float32)
                    + bc1_ref[...], 0.0)
    o_ref[...] = jnp.dot(c, wc2_ref[...], preferred_element_type=jnp.float32)


def _gru_head(gin, wih_t, whh_t, bih, bhh, wc1, bc1, wc2p):
    rb = 256
    grid = B // rb
    out = pl.pallas_call(
        _gru_kernel,
        grid=(grid,),
        in_specs=[
            pl.BlockSpec((rb, 3 * H), lambda i: (i, 0)),
            pl.BlockSpec((rb, 3 * H), lambda i: (i + grid, 0)),
            pl.BlockSpec((3 * H, 3 * H), lambda i: (0, 0)),
            pl.BlockSpec((H, 3 * H), lambda i: (0, 0)),
            pl.BlockSpec((1, 3 * H), lambda i: (0, 0)),
            pl.BlockSpec((1, 3 * H), lambda i: (0, 0)),
            pl.BlockSpec((H, 2 * H), lambda i: (0, 0)),
            pl.BlockSpec((1, 2 * H), lambda i: (0, 0)),
            pl.BlockSpec((2 * H, 128), lambda i: (0, 0)),
        ],
        out_specs=pl.BlockSpec((rb, 128), lambda i: (i, 0)),
        out_shape=jax.ShapeDtypeStruct((B, 128), jnp.float32),
    )(gin, gin, wih_t, whh_t, bih, bhh, wc1, bc1, wc2p)
    return out[:, :1]


def kernel(x_batch, LOS_batch, template_edge_index, emb_table, W1a, b1a,
           ln1_g, ln1_b, W1b, b1b, W2a, b2a, ln2_g, ln2_b, W2b, b2b,
           Wih, Whh, bih, bhh, Wc1, bc1, Wc2, bc2):
    g = jnp.arange(NGRAPH, dtype=jnp.int32)
    is_dis = (g >= B).astype(jnp.int32)
    brow = g - is_dis * B
    cols = is_dis[:, None] * 36 + jnp.arange(N, dtype=jnp.int32)[None, :]
    flat = x_batch[brow[:, None], cols] + cols * COL_DIM
    nidx = flat.reshape(NODES)
    los_node = jnp.broadcast_to(
        (LOS_batch[brow] / MAX_LOS)[:, None], (NGRAPH, N)).reshape(NODES)

    src2 = template_edge_index[0].astype(jnp.int32).reshape(32, EPW)
    dst2 = template_edge_index[1].astype(jnp.int32).reshape(32, EPW)
    pbkt, nblk = _bucketize(src2, dst2)

    emb_h = _emb_gather(emb_table, nidx.astype(jnp.int32).reshape(32, 18, 128))

    pool_mat = jnp.repeat(jnp.eye(GB, dtype=jnp.float32), N, axis=1)

    biases2 = [(b1a, ln1_g, ln1_b, W1b, b1b),
               (b2a, ln2_g, ln2_b, W2b, b2b),
               (b2a, ln2_g, ln2_b, W2b, b2b)]
    pooled = []
    h = None
    for li in range(3):
        if li == 0:
            y = _matmul1(emb_h, los_node[:, None], W1a[:EMB], W1a[EMB:])
        else:
            y = _matmul(h, W2a)
        agg = _edge_agg(y, pbkt, nblk)
        ba, lg, lb, wb, bb = biases2[li]
        h, p = _mlpb(y, agg, ba.reshape(1, H), lg.reshape(1, H),
                     lb.reshape(1, H), wb, bb.reshape(1, H), pool_mat)
        pooled.append(p)
    gin = jnp.concatenate(pooled, axis=1)

    wc2p = jnp.concatenate(
        [Wc2, jnp.zeros((2 * H, 127), jnp.float32)], axis=1)
    out = _gru_head(gin, Wih.T, Whh.T, bih.reshape(1, 3 * H),
                    bhh.reshape(1, 3 * H), Wc1, bc1.reshape(1, 2 * H), wc2p)
    return out + bc2[0]
